# Initial kernel scaffold; baseline (speedup 1.0000x reference)
import jax
import jax.numpy as jnp
from jax.experimental import pallas as pl


def kernel(X, attn_mask, Wq, bq, Wk, bk, Wv, bv):
    raise NotImplementedError("write your pallas kernel here")


# speedup vs baseline: 29.9754x; 29.9754x over previous
from math import sqrt

import jax
import jax.numpy as jnp
import numpy as np
from jax import lax
from jax.experimental import pallas as pl
from jax.experimental.pallas import tpu as pltpu

_SAMPLE_FACTOR = 10
_NEG = -1e30

_cnt_cache = {}


def _sample_cntT(L_Q, L_K, U_part):
    cache_key = (L_Q, L_K, U_part)
    if cache_key not in _cnt_cache:
        with jax.ensure_compile_time_eval():
            idx = np.asarray(
                jax.random.randint(jax.random.key(42), (L_Q, U_part), 0, L_K)
            )
        cntT = np.zeros((L_K, L_Q), np.float32)
        np.add.at(cntT, (idx.ravel(), np.repeat(np.arange(L_Q), U_part)), 1.0)
        _cnt_cache[cache_key] = cntT
    return _cnt_cache[cache_key]


def _attn_body(x_ref, wq_ref, bq_ref, wk_ref, bk_ref, wv_ref, bv_ref,
               cntT_ref, out_ref, q_s, k_s, v_s, p_s, *, L, D, u, kb):
    x = x_ref[0, 0]
    dn_t = (((1,), (1,)), ((), ()))

    q_s[...] = jnp.maximum(
        lax.dot_general(x, wq_ref[...], dn_t, preferred_element_type=jnp.float32)
        + bq_ref[...], 0.0)
    k_s[...] = jnp.maximum(
        lax.dot_general(x, wk_ref[...], dn_t, preferred_element_type=jnp.float32)
        + bk_ref[...], 0.0)
    v_s[...] = jnp.maximum(
        lax.dot_general(x, wv_ref[...], dn_t, preferred_element_type=jnp.float32)
        + bv_ref[...], 0.0)

    q = q_s[...]

    mx = jnp.full((1, L), _NEG, jnp.float32)
    sm = jnp.zeros((1, L), jnp.float32)
    for b in range(L // kb):
        kblk = k_s[b * kb:(b + 1) * kb, :]
        st = lax.dot_general(kblk, q, dn_t, preferred_element_type=jnp.float32)
        cb = cntT_ref[b * kb:(b + 1) * kb, :]
        mx = jnp.maximum(mx, jnp.max(jnp.where(cb > 0.0, st, _NEG), axis=0,
                                     keepdims=True))
        sm = sm + jnp.sum(st * cb, axis=0, keepdims=True)
    m_meas = mx - sm * (1.0 / L)

    iot = lax.broadcasted_iota(jnp.int32, (1, L), 1)

    def sel_body(i, m):
        mval = jnp.max(m)
        idx = jnp.min(jnp.where(m == mval, iot, L))
        oh = iot == idx
        p_s[pl.ds(i, 1), :] = oh.astype(jnp.float32)
        return jnp.where(oh, _NEG, m)

    lax.fori_loop(0, u, sel_body, m_meas)

    p = p_s[...]
    qr = jnp.dot(p, q, preferred_element_type=jnp.float32)
    scores = lax.dot_general(qr, k_s[...], dn_t,
                             preferred_element_type=jnp.float32) * (1.0 / sqrt(D))
    scores = scores - jnp.max(scores, axis=1, keepdims=True)
    e = jnp.exp(scores)
    attn = e / jnp.sum(e, axis=1, keepdims=True)
    upd = jnp.dot(attn, v_s[...], preferred_element_type=jnp.float32)

    vmean = jnp.mean(v_s[...], axis=0, keepdims=True)
    dn_lt = (((0,), (0,)), ((), ()))
    ctx = lax.dot_general(p, upd, dn_lt, preferred_element_type=jnp.float32)
    ind = lax.dot_general(p, jnp.ones((u, 1), jnp.float32), dn_lt,
                          preferred_element_type=jnp.float32)
    out = vmean * (1.0 - ind) + ctx
    out_ref[...] = out.reshape(1, 1, L, D)


def kernel(X, attn_mask, Wq, bq, Wk, bk, Wv, bv):
    B, H, L_Q, D = X.shape
    L_K = L_Q
    U_part = min(_SAMPLE_FACTOR * int(np.ceil(np.log(L_K))), L_K)
    u = min(_SAMPLE_FACTOR * int(np.ceil(np.log(L_Q))), L_Q)

    cntT = jnp.asarray(_sample_cntT(L_Q, L_K, U_part))

    import functools
    body = functools.partial(_attn_body, L=L_Q, D=D, u=u, kb=256)

    rep = lambda b, h: (0, 0)
    out = pl.pallas_call(
        body,
        grid=(B, H),
        in_specs=[
            pl.BlockSpec((1, 1, L_Q, D), lambda b, h: (b, h, 0, 0)),
            pl.BlockSpec((D, D), rep),
            pl.BlockSpec((1, D), rep),
            pl.BlockSpec((D, D), rep),
            pl.BlockSpec((1, D), rep),
            pl.BlockSpec((D, D), rep),
            pl.BlockSpec((1, D), rep),
            pl.BlockSpec((L_K, L_Q), rep),
        ],
        out_specs=pl.BlockSpec((1, 1, L_Q, D), lambda b, h: (b, h, 0, 0)),
        out_shape=jax.ShapeDtypeStruct((B, H, L_Q, D), jnp.float32),
        scratch_shapes=[
            pltpu.VMEM((L_Q, D), jnp.float32),
            pltpu.VMEM((L_K, D), jnp.float32),
            pltpu.VMEM((L_K, D), jnp.float32),
            pltpu.VMEM((u, L_K), jnp.float32),
        ],
    )(X, Wq, bq.reshape(1, D), Wk, bk.reshape(1, D), Wv, bv.reshape(1, D), cntT)
    return jnp.swapaxes(out, 1, 2)


# speedup vs baseline: 78.2165x; 2.6094x over previous
from math import sqrt

import jax
import jax.numpy as jnp
import numpy as np
from jax import lax
from jax.experimental import pallas as pl
from jax.experimental.pallas import tpu as pltpu

_SAMPLE_FACTOR = 10
_NEG = -1e30

_cnt_cache = {}


def _threefry2x32(k0, k1, x0, x1):
    rot_a = (13, 15, 26, 6)
    rot_b = (17, 29, 16, 24)
    ks0 = np.uint32(k0)
    ks1 = np.uint32(k1)
    ks2 = np.uint32(ks0 ^ ks1 ^ np.uint32(0x1BD11BDA))
    x0 = (x0 + ks0).astype(np.uint32)
    x1 = (x1 + ks1).astype(np.uint32)

    def rnd(x0, x1, r):
        x0 = (x0 + x1).astype(np.uint32)
        x1 = ((x1 << np.uint32(r)) | (x1 >> np.uint32(32 - r))).astype(np.uint32)
        return x0, x1 ^ x0

    for i, (ka, kb, rots) in enumerate(
        ((ks1, ks2, rot_a), (ks2, ks0, rot_b), (ks0, ks1, rot_a),
         (ks1, ks2, rot_b), (ks2, ks0, rot_a))):
        for r in rots:
            x0, x1 = rnd(x0, x1, r)
        x0 = (x0 + ka).astype(np.uint32)
        x1 = (x1 + kb + np.uint32(i + 1)).astype(np.uint32)
    return x0, x1


def _np_randint_key42(shape, maxval):
    n = int(np.prod(shape))

    def bits(k0, k1, m):
        b0, b1 = _threefry2x32(k0, k1, np.zeros(m, np.uint32),
                               np.arange(m, dtype=np.uint32))
        return b0 ^ b1

    s0, s1 = _threefry2x32(np.uint32(0), np.uint32(42),
                           np.zeros(2, np.uint32),
                           np.arange(2, dtype=np.uint32))
    hi = bits(s0[0], s1[0], n)
    lo = bits(s0[1], s1[1], n)
    span = np.uint32(maxval)
    mult = np.uint32(((2 ** 16) % maxval) ** 2 % maxval)
    off = ((hi % span) * mult + lo % span) % span
    return off.astype(np.int32).reshape(shape)


def _sample_cntT(L_Q, L_K, U_part):
    cache_key = (L_Q, L_K, U_part)
    if cache_key not in _cnt_cache:
        idx = _np_randint_key42((L_Q, U_part), L_K)
        cntT = np.zeros((L_K, L_Q), np.float32)
        np.add.at(cntT, (idx.ravel(), np.repeat(np.arange(L_Q), U_part)), 1.0)
        _cnt_cache[cache_key] = cntT
    return _cnt_cache[cache_key]


def _attn_body(x_ref, wq_ref, bq_ref, wk_ref, bk_ref, wv_ref, bv_ref,
               cntT_ref, out_ref, q_s, k_s, v_s, *, L, D, u, kb):
    x = x_ref[0, 0]
    dn_t = (((1,), (1,)), ((), ()))

    q_s[...] = jnp.maximum(
        lax.dot_general(x, wq_ref[...], dn_t, preferred_element_type=jnp.float32)
        + bq_ref[...], 0.0)
    k_s[...] = jnp.maximum(
        lax.dot_general(x, wk_ref[...], dn_t, preferred_element_type=jnp.float32)
        + bk_ref[...], 0.0)
    v_s[...] = jnp.maximum(
        lax.dot_general(x, wv_ref[...], dn_t, preferred_element_type=jnp.float32)
        + bv_ref[...], 0.0)

    q = q_s[...]

    mx = jnp.full((1, L), _NEG, jnp.float32)
    sm = jnp.zeros((1, L), jnp.float32)
    for b in range(L // kb):
        kblk = k_s[b * kb:(b + 1) * kb, :]
        st = lax.dot_general(kblk, q, dn_t, preferred_element_type=jnp.float32)
        cb = cntT_ref[b * kb:(b + 1) * kb, :]
        mx = jnp.maximum(mx, jnp.max(jnp.where(cb > 0.0, st, _NEG), axis=0,
                                     keepdims=True))
        sm = sm + jnp.sum(st * cb, axis=0, keepdims=True)
    m_meas = mx - sm * (1.0 / L)

    dn_lt = (((0,), (0,)), ((), ()))
    m_col = jnp.reshape(m_meas, (L, 1))
    rank = jnp.zeros((1, L), jnp.float32)
    for b in range(L // kb):
        mk = m_col[b * kb:(b + 1) * kb, :]
        io_k = lax.broadcasted_iota(jnp.int32, (kb, L), 0) + (b * kb)
        io_q = lax.broadcasted_iota(jnp.int32, (kb, L), 1)
        wins = (mk > m_meas) | ((mk == m_meas) & (io_k < io_q))
        rank = rank + jnp.sum(jnp.where(wins, 1.0, 0.0), axis=0,
                              keepdims=True)

    slot = lax.broadcasted_iota(jnp.int32, (u, L), 0).astype(jnp.float32)
    p = jnp.where(rank == slot, 1.0, 0.0)

    qr = jnp.dot(p, q, preferred_element_type=jnp.float32)
    scores = lax.dot_general(qr, k_s[...], dn_t,
                             preferred_element_type=jnp.float32) * (1.0 / sqrt(D))
    scores = scores - jnp.max(scores, axis=1, keepdims=True)
    e = jnp.exp(scores)
    attn = e / jnp.sum(e, axis=1, keepdims=True)
    upd = jnp.dot(attn, v_s[...], preferred_element_type=jnp.float32)

    vmean = jnp.mean(v_s[...], axis=0, keepdims=True)
    ctx = lax.dot_general(p, upd, dn_lt, preferred_element_type=jnp.float32)
    ind = lax.dot_general(p, jnp.ones((u, 1), jnp.float32), dn_lt,
                          preferred_element_type=jnp.float32)
    out = jnp.where(ind > 0.5, ctx, vmean)
    out_ref[...] = out.reshape(1, 1, L, D)


def kernel(X, attn_mask, Wq, bq, Wk, bk, Wv, bv):
    B, H, L_Q, D = X.shape
    L_K = L_Q
    U_part = min(_SAMPLE_FACTOR * int(np.ceil(np.log(L_K))), L_K)
    u = min(_SAMPLE_FACTOR * int(np.ceil(np.log(L_Q))), L_Q)

    cntT = jnp.asarray(_sample_cntT(L_Q, L_K, U_part))

    import functools
    body = functools.partial(_attn_body, L=L_Q, D=D, u=u, kb=256)

    rep = lambda b, h: (0, 0)
    out = pl.pallas_call(
        body,
        grid=(B, H),
        in_specs=[
            pl.BlockSpec((1, 1, L_Q, D), lambda b, h: (b, h, 0, 0)),
            pl.BlockSpec((D, D), rep),
            pl.BlockSpec((1, D), rep),
            pl.BlockSpec((D, D), rep),
            pl.BlockSpec((1, D), rep),
            pl.BlockSpec((D, D), rep),
            pl.BlockSpec((1, D), rep),
            pl.BlockSpec((L_K, L_Q), rep),
        ],
        out_specs=pl.BlockSpec((1, 1, L_Q, D), lambda b, h: (b, h, 0, 0)),
        out_shape=jax.ShapeDtypeStruct((B, H, L_Q, D), jnp.float32),
        scratch_shapes=[
            pltpu.VMEM((L_Q, D), jnp.float32),
            pltpu.VMEM((L_K, D), jnp.float32),
            pltpu.VMEM((L_K, D), jnp.float32),
        ],
    )(X, Wq, bq.reshape(1, D), Wk, bk.reshape(1, D), Wv, bv.reshape(1, D), cntT)
    return jnp.swapaxes(out, 1, 2)


# speedup vs baseline: 87.2344x; 1.1153x over previous
from math import sqrt

import jax
import jax.numpy as jnp
import numpy as np
from jax import lax
from jax.experimental import pallas as pl
from jax.experimental.pallas import tpu as pltpu

_SAMPLE_FACTOR = 10
_NEG = -1e30

_cnt_cache = {}


def _threefry2x32(k0, k1, x0, x1):
    rot_a = (13, 15, 26, 6)
    rot_b = (17, 29, 16, 24)
    ks0 = np.uint32(k0)
    ks1 = np.uint32(k1)
    ks2 = np.uint32(ks0 ^ ks1 ^ np.uint32(0x1BD11BDA))
    x0 = (x0 + ks0).astype(np.uint32)
    x1 = (x1 + ks1).astype(np.uint32)

    def rnd(x0, x1, r):
        x0 = (x0 + x1).astype(np.uint32)
        x1 = ((x1 << np.uint32(r)) | (x1 >> np.uint32(32 - r))).astype(np.uint32)
        return x0, x1 ^ x0

    for i, (ka, kb, rots) in enumerate(
        ((ks1, ks2, rot_a), (ks2, ks0, rot_b), (ks0, ks1, rot_a),
         (ks1, ks2, rot_b), (ks2, ks0, rot_a))):
        for r in rots:
            x0, x1 = rnd(x0, x1, r)
        x0 = (x0 + ka).astype(np.uint32)
        x1 = (x1 + kb + np.uint32(i + 1)).astype(np.uint32)
    return x0, x1


def _np_randint_key42(shape, maxval):
    n = int(np.prod(shape))

    def bits(k0, k1, m):
        b0, b1 = _threefry2x32(k0, k1, np.zeros(m, np.uint32),
                               np.arange(m, dtype=np.uint32))
        return b0 ^ b1

    s0, s1 = _threefry2x32(np.uint32(0), np.uint32(42),
                           np.zeros(2, np.uint32),
                           np.arange(2, dtype=np.uint32))
    hi = bits(s0[0], s1[0], n)
    lo = bits(s0[1], s1[1], n)
    span = np.uint32(maxval)
    mult = np.uint32(((2 ** 16) % maxval) ** 2 % maxval)
    off = ((hi % span) * mult + lo % span) % span
    return off.astype(np.int32).reshape(shape)


def _sample_cntT(L_Q, L_K, U_part):
    cache_key = (L_Q, L_K, U_part)
    if cache_key not in _cnt_cache:
        idx = _np_randint_key42((L_Q, U_part), L_K)
        cntT = np.zeros((L_K, L_Q), np.float32)
        np.add.at(cntT, (idx.ravel(), np.repeat(np.arange(L_Q), U_part)), 1.0)
        _cnt_cache[cache_key] = cntT
    return _cnt_cache[cache_key]


def _attn_body(x_ref, wq_ref, bq_ref, wk_ref, bk_ref, wv_ref, bv_ref,
               cntT_ref, out_ref, q_s, k_s, v_s, *, L, D, H, u, kb):
    for h in range(H):
        out_h = _one_head(x_ref[0, h], wq_ref, bq_ref, wk_ref, bk_ref,
                          wv_ref, bv_ref, cntT_ref, q_s, k_s, v_s,
                          L=L, D=D, u=u, kb=kb)
        out_ref[0, :, h * D:(h + 1) * D] = out_h


def _one_head(x, wq_ref, bq_ref, wk_ref, bk_ref, wv_ref, bv_ref,
              cntT_ref, q_s, k_s, v_s, *, L, D, u, kb):
    dn_t = (((1,), (1,)), ((), ()))

    q_s[...] = jnp.maximum(
        lax.dot_general(x, wq_ref[...], dn_t, preferred_element_type=jnp.float32)
        + bq_ref[...], 0.0)
    k_s[...] = jnp.maximum(
        lax.dot_general(x, wk_ref[...], dn_t, preferred_element_type=jnp.float32)
        + bk_ref[...], 0.0)
    v_s[...] = jnp.maximum(
        lax.dot_general(x, wv_ref[...], dn_t, preferred_element_type=jnp.float32)
        + bv_ref[...], 0.0)

    q = q_s[...]

    mx = jnp.full((1, L), _NEG, jnp.float32)
    sm = jnp.zeros((1, L), jnp.float32)
    for b in range(L // kb):
        kblk = k_s[b * kb:(b + 1) * kb, :]
        st = lax.dot_general(kblk, q, dn_t, preferred_element_type=jnp.float32)
        cb = cntT_ref[b * kb:(b + 1) * kb, :].astype(jnp.float32)
        mx = jnp.maximum(mx, jnp.max(jnp.where(cb > 0.0, st, _NEG), axis=0,
                                     keepdims=True))
        sm = sm + jnp.sum(st * cb, axis=0, keepdims=True)
    m_meas = mx - sm * (1.0 / L)

    dn_lt = (((0,), (0,)), ((), ()))
    m_col = jnp.reshape(m_meas, (L, 1))
    rank = jnp.zeros((1, L), jnp.float32)
    for b in range(L // kb):
        mk = m_col[b * kb:(b + 1) * kb, :]
        io_k = lax.broadcasted_iota(jnp.int32, (kb, L), 0) + (b * kb)
        io_q = lax.broadcasted_iota(jnp.int32, (kb, L), 1)
        wins = (mk > m_meas) | ((mk == m_meas) & (io_k < io_q))
        rank = rank + jnp.sum(jnp.where(wins, 1.0, 0.0), axis=0,
                              keepdims=True)

    slot = lax.broadcasted_iota(jnp.int32, (u, L), 0).astype(jnp.float32)
    p = jnp.where(rank == slot, 1.0, 0.0)

    qr = jnp.dot(p, q, preferred_element_type=jnp.float32)
    scores = lax.dot_general(qr, k_s[...], dn_t,
                             preferred_element_type=jnp.float32) * (1.0 / sqrt(D))
    scores = scores - jnp.max(scores, axis=1, keepdims=True)
    e = jnp.exp(scores)
    attn = e / jnp.sum(e, axis=1, keepdims=True)
    upd = jnp.dot(attn, v_s[...], preferred_element_type=jnp.float32)

    vmean = jnp.mean(v_s[...], axis=0, keepdims=True)
    ctx = lax.dot_general(p, upd, dn_lt, preferred_element_type=jnp.float32)
    ind = lax.dot_general(p, jnp.ones((u, 1), jnp.float32), dn_lt,
                          preferred_element_type=jnp.float32)
    return jnp.where(ind > 0.5, ctx, vmean)


def kernel(X, attn_mask, Wq, bq, Wk, bk, Wv, bv):
    B, H, L_Q, D = X.shape
    L_K = L_Q
    U_part = min(_SAMPLE_FACTOR * int(np.ceil(np.log(L_K))), L_K)
    u = min(_SAMPLE_FACTOR * int(np.ceil(np.log(L_Q))), L_Q)

    cntT = jnp.asarray(_sample_cntT(L_Q, L_K, U_part)).astype(jnp.bfloat16)

    import functools
    body = functools.partial(_attn_body, L=L_Q, D=D, H=H, u=u, kb=256)

    rep = lambda b: (0, 0)
    out = pl.pallas_call(
        body,
        grid=(B,),
        in_specs=[
            pl.BlockSpec((1, H, L_Q, D), lambda b: (b, 0, 0, 0)),
            pl.BlockSpec((D, D), rep),
            pl.BlockSpec((1, D), rep),
            pl.BlockSpec((D, D), rep),
            pl.BlockSpec((1, D), rep),
            pl.BlockSpec((D, D), rep),
            pl.BlockSpec((1, D), rep),
            pl.BlockSpec((L_K, L_Q), rep),
        ],
        out_specs=pl.BlockSpec((1, L_Q, H * D), lambda b: (b, 0, 0)),
        out_shape=jax.ShapeDtypeStruct((B, L_Q, H * D), jnp.float32),
        scratch_shapes=[
            pltpu.VMEM((L_Q, D), jnp.float32),
            pltpu.VMEM((L_K, D), jnp.float32),
            pltpu.VMEM((L_K, D), jnp.float32),
        ],
    )(X, Wq, bq.reshape(1, D), Wk, bk.reshape(1, D), Wv, bv.reshape(1, D), cntT)
    return out.reshape(B, L_Q, H, D)


# speedup vs baseline: 131.6423x; 1.5091x over previous
from math import sqrt

import jax
import jax.numpy as jnp
import numpy as np
from jax import lax
from jax.experimental import pallas as pl
from jax.experimental.pallas import tpu as pltpu

_SAMPLE_FACTOR = 10
_NEG = -1e30

_cnt_cache = {}


def _threefry2x32(k0, k1, x0, x1):
    rot_a = (13, 15, 26, 6)
    rot_b = (17, 29, 16, 24)
    ks0 = np.uint32(k0)
    ks1 = np.uint32(k1)
    ks2 = np.uint32(ks0 ^ ks1 ^ np.uint32(0x1BD11BDA))
    x0 = (x0 + ks0).astype(np.uint32)
    x1 = (x1 + ks1).astype(np.uint32)

    def rnd(x0, x1, r):
        x0 = (x0 + x1).astype(np.uint32)
        x1 = ((x1 << np.uint32(r)) | (x1 >> np.uint32(32 - r))).astype(np.uint32)
        return x0, x1 ^ x0

    for i, (ka, kb, rots) in enumerate(
        ((ks1, ks2, rot_a), (ks2, ks0, rot_b), (ks0, ks1, rot_a),
         (ks1, ks2, rot_b), (ks2, ks0, rot_a))):
        for r in rots:
            x0, x1 = rnd(x0, x1, r)
        x0 = (x0 + ka).astype(np.uint32)
        x1 = (x1 + kb + np.uint32(i + 1)).astype(np.uint32)
    return x0, x1


def _np_randint_key42(shape, maxval):
    n = int(np.prod(shape))

    def bits(k0, k1, m):
        b0, b1 = _threefry2x32(k0, k1, np.zeros(m, np.uint32),
                               np.arange(m, dtype=np.uint32))
        return b0 ^ b1

    s0, s1 = _threefry2x32(np.uint32(0), np.uint32(42),
                           np.zeros(2, np.uint32),
                           np.arange(2, dtype=np.uint32))
    hi = bits(s0[0], s1[0], n)
    lo = bits(s0[1], s1[1], n)
    span = np.uint32(maxval)
    mult = np.uint32(((2 ** 16) % maxval) ** 2 % maxval)
    off = ((hi % span) * mult + lo % span) % span
    return off.astype(np.int32).reshape(shape)


def _sample_cntT(L_Q, L_K, U_part):
    cache_key = (L_Q, L_K, U_part)
    if cache_key not in _cnt_cache:
        idx = _np_randint_key42((L_Q, U_part), L_K)
        cntT = np.zeros((L_K, L_Q), np.float32)
        np.add.at(cntT, (idx.ravel(), np.repeat(np.arange(L_Q), U_part)), 1.0)
        _cnt_cache[cache_key] = cntT
    return _cnt_cache[cache_key]


def _attn_body(x_ref, wq_ref, bq_ref, wk_ref, bk_ref, wv_ref, bv_ref,
               cntT_ref, out_ref, q_s, k_s, v_s, *, L, D, H, u, kb):
    for h in range(H):
        out_t = _one_head(x_ref[0, h], wq_ref, bq_ref, wk_ref, bk_ref,
                          wv_ref, bv_ref, cntT_ref, q_s, k_s, v_s,
                          L=L, D=D, u=u, kb=kb)
        out_ref[0, h * D:(h + 1) * D, :] = out_t


def _one_head(xt, wq_ref, bq_ref, wk_ref, bk_ref, wv_ref, bv_ref,
              cntT_ref, q_s, k_s, v_s, *, L, D, u, kb):
    dn_nn = (((1,), (0,)), ((), ()))
    dn_tn = (((0,), (0,)), ((), ()))
    dn_nt = (((1,), (1,)), ((), ()))

    bq_col = jnp.reshape(bq_ref[...], (D, 1))
    bk_col = jnp.reshape(bk_ref[...], (D, 1))
    bv_col = jnp.reshape(bv_ref[...], (D, 1))
    q_s[...] = jnp.maximum(
        lax.dot_general(wq_ref[...], xt, dn_nn,
                        preferred_element_type=jnp.float32) + bq_col, 0.0)
    k_s[...] = jnp.maximum(
        lax.dot_general(wk_ref[...], xt, dn_nn,
                        preferred_element_type=jnp.float32) + bk_col, 0.0)
    v_s[...] = jnp.maximum(
        lax.dot_general(wv_ref[...], xt, dn_nn,
                        preferred_element_type=jnp.float32) + bv_col, 0.0)

    qt = q_s[...]

    mx = jnp.full((1, L), _NEG, jnp.float32)
    sm = jnp.zeros((1, L), jnp.float32)
    for b in range(L // kb):
        kblk = k_s[:, b * kb:(b + 1) * kb]
        st = lax.dot_general(kblk, qt, dn_tn,
                             preferred_element_type=jnp.float32)
        cb = cntT_ref[b * kb:(b + 1) * kb, :]
        mx = jnp.maximum(mx, jnp.max(jnp.where(cb > 0.0, st, _NEG), axis=0,
                                     keepdims=True))
        sm = sm + jnp.sum(st * cb, axis=0, keepdims=True)
    m_meas = mx - sm * (1.0 / L)

    m_col = jnp.reshape(m_meas, (L, 1))
    rank = jnp.zeros((1, L), jnp.float32)
    for b in range(L // kb):
        mk = m_col[b * kb:(b + 1) * kb, :]
        io_k = lax.broadcasted_iota(jnp.int32, (kb, L), 0) + (b * kb)
        io_q = lax.broadcasted_iota(jnp.int32, (kb, L), 1)
        wins = (mk > m_meas) | ((mk == m_meas) & (io_k < io_q))
        rank = rank + jnp.sum(jnp.where(wins, 1.0, 0.0), axis=0,
                              keepdims=True)

    slot = lax.broadcasted_iota(jnp.int32, (u, L), 0).astype(jnp.float32)
    p = jnp.where(rank == slot, 1.0, 0.0)

    qr = lax.dot_general(p, qt, dn_nt,
                         preferred_element_type=jnp.float32)
    scores = lax.dot_general(qr, k_s[...], dn_nn,
                             preferred_element_type=jnp.float32) * (1.0 / sqrt(D))
    scores = scores - jnp.max(scores, axis=1, keepdims=True)
    e = jnp.exp(scores)
    attn = e / jnp.sum(e, axis=1, keepdims=True)
    upd = lax.dot_general(attn, v_s[...], dn_nt,
                          preferred_element_type=jnp.float32)

    vmean_col = jnp.mean(v_s[...], axis=1, keepdims=True)
    ctx_t = lax.dot_general(upd, p, dn_tn,
                            preferred_element_type=jnp.float32)
    ind = jnp.sum(p, axis=0, keepdims=True)
    return jnp.where(ind > 0.5, ctx_t, vmean_col)


def kernel(X, attn_mask, Wq, bq, Wk, bk, Wv, bv):
    B, H, L_Q, D = X.shape
    L_K = L_Q
    U_part = min(_SAMPLE_FACTOR * int(np.ceil(np.log(L_K))), L_K)
    u = min(_SAMPLE_FACTOR * int(np.ceil(np.log(L_Q))), L_Q)

    cntT = jnp.asarray(_sample_cntT(L_Q, L_K, U_part))

    import functools
    body = functools.partial(_attn_body, L=L_Q, D=D, H=H, u=u, kb=256)

    rep = lambda b: (0, 0)
    xt = jnp.swapaxes(X, 2, 3)
    out = pl.pallas_call(
        body,
        grid=(B,),
        in_specs=[
            pl.BlockSpec((1, H, D, L_Q), lambda b: (b, 0, 0, 0)),
            pl.BlockSpec((D, D), rep),
            pl.BlockSpec((1, D), rep),
            pl.BlockSpec((D, D), rep),
            pl.BlockSpec((1, D), rep),
            pl.BlockSpec((D, D), rep),
            pl.BlockSpec((1, D), rep),
            pl.BlockSpec((L_K, L_Q), rep),
        ],
        out_specs=pl.BlockSpec((1, H * D, L_Q), lambda b: (b, 0, 0)),
        out_shape=jax.ShapeDtypeStruct((B, H * D, L_Q), jnp.float32),
        scratch_shapes=[
            pltpu.VMEM((D, L_Q), jnp.float32),
            pltpu.VMEM((D, L_K), jnp.float32),
            pltpu.VMEM((D, L_K), jnp.float32),
        ],
    )(xt, Wq, bq.reshape(1, D), Wk, bk.reshape(1, D), Wv, bv.reshape(1, D),
      cntT)
    return jnp.transpose(out.reshape(B, H, D, L_Q), (0, 3, 1, 2))
